# baseline (speedup 1.0000x reference)
import functools

import jax
import jax.numpy as jnp
from jax import lax
from jax.experimental import pallas as pl
from jax.experimental.pallas import tpu as pltpu
from jax.experimental.pallas import tpu_sc as plsc

N = 10000
E = 320000
DIN = 128
DH = 32
DOUT = 32
TOPK = 16
NP0 = DIN * DH + DH
NP1 = DH * DOUT + DOUT

NC = 2
NS = 16
NW = NC * NS
NPAD = 10240
SLICE = NPAD // NS
E_PAD = 327680
E_PER_W = E_PAD // NW
CHUNK = 2048
N_CH = E_PER_W // CHUNK
SCHUNK = 1280
S_CH = E_PER_W // SCHUNK

_f32 = jnp.float32


def _mesh():
    return plsc.VectorSubcoreMesh(
        core_axis_name="c", subcore_axis_name="s", num_cores=NC, num_subcores=NS
    )


_SC_PARAMS = pltpu.CompilerParams(use_tc_tiling_on_sc=False, needs_layout_passes=False)


_Z16 = None


def _deg_body(dst_hbm, out_hbm, idx_v, hist_v):
    c = lax.axis_index("c")
    s = lax.axis_index("s")
    wid = s * NC + c
    zeros16 = jnp.zeros((16,), _f32)
    ones16 = jnp.ones((16,), _f32)

    @plsc.parallel_loop(0, NPAD, 16, unroll=8)
    def zbody(i):
        hist_v[pl.ds(i, 16)] = zeros16

    for k in range(N_CH):
        base = pl.multiple_of(wid * E_PER_W + k * CHUNK, 8)
        pltpu.sync_copy(dst_hbm.at[pl.ds(base, CHUNK)], idx_v)

        @plsc.parallel_loop(0, CHUNK, 16, unroll=8)
        def ibody(i):
            d = idx_v[pl.ds(i, 16)]
            plsc.addupdate_scatter(hist_v, [d], ones16)

    pltpu.sync_copy(hist_v, out_hbm.at[wid])


def _degree(dst_p):
    f = pl.kernel(
        _deg_body,
        out_type=jax.ShapeDtypeStruct((NW, NPAD), _f32),
        mesh=_mesh(),
        scratch_types=[
            pltpu.VMEM((CHUNK,), jnp.int32),
            pltpu.VMEM((NPAD,), _f32),
        ],
        compiler_params=_SC_PARAMS,
    )
    return f(dst_p)


DT = 4
NM = 4
E_QTR = E_PAD // NM
ECH = 8192
E_NCH = E_QTR // ECH


def _edge_body(src_hbm, dst_hbm, xwst_hbm, out_hbm,
               sidx_a, sidx_b, didx_a, didx_b,
               feat0, feat1, feat2, feat3, acc0, acc1, acc2, acc3,
               sem0, sem1, sem2, sem3):
    c = lax.axis_index("c")
    s = lax.axis_index("s")
    g = s & 7
    m = (s >> 3) * 2 + c
    zeros16 = jnp.zeros((16,), _f32)
    sidxs = (sidx_a, sidx_b)
    didxs = (didx_a, didx_b)
    feats = (feat0, feat1, feat2, feat3)
    accs = (acc0, acc1, acc2, acc3)
    sems = ((sem0, sem1), (sem2, sem3))

    def load(k, b):
        base = pl.multiple_of(m * E_QTR + k * ECH, 8)
        d1 = pltpu.async_copy(src_hbm.at[pl.ds(base, ECH)], sidxs[b], sems[b][0])
        d2 = pltpu.async_copy(dst_hbm.at[pl.ds(base, ECH)], didxs[b], sems[b][1])
        return (d1, d2)

    descs = [load(0, 0), None]

    @plsc.parallel_loop(0, NPAD, 16, unroll=8)
    def zbody(i):
        for a in accs:
            a[pl.ds(i, 16)] = zeros16

    for j in range(DT):
        pltpu.sync_copy(xwst_hbm.at[DT * g + j], feats[j])
    for k in range(E_NCH):
        b = k & 1
        if k + 1 < E_NCH:
            descs[1 - b] = load(k + 1, 1 - b)
        for d in descs[b]:
            d.wait()
        sidx_v = sidxs[b]
        didx_v = didxs[b]

        @plsc.parallel_loop(0, ECH, 16, unroll=16)
        def ibody(i):
            sv = sidx_v[pl.ds(i, 16)]
            dv = didx_v[pl.ds(i, 16)]
            for f, a in zip(feats, accs):
                v = plsc.load_gather(f, [sv])
                plsc.addupdate_scatter(a, [dv], v)

    for j in range(DT):
        pltpu.sync_copy(accs[j], out_hbm.at[m, DT * g + j])


def _edge_scatter(src_p, dst_p, xwst):
    f = pl.kernel(
        _edge_body,
        out_type=jax.ShapeDtypeStruct((NM, DH, NPAD), _f32),
        mesh=_mesh(),
        scratch_types=[
            pltpu.VMEM((ECH,), jnp.int32),
            pltpu.VMEM((ECH,), jnp.int32),
            pltpu.VMEM((ECH,), jnp.int32),
            pltpu.VMEM((ECH,), jnp.int32),
            pltpu.VMEM((NPAD,), _f32),
            pltpu.VMEM((NPAD,), _f32),
            pltpu.VMEM((NPAD,), _f32),
            pltpu.VMEM((NPAD,), _f32),
            pltpu.VMEM((NPAD,), _f32),
            pltpu.VMEM((NPAD,), _f32),
            pltpu.VMEM((NPAD,), _f32),
            pltpu.VMEM((NPAD,), _f32),
            pltpu.SemaphoreType.DMA,
            pltpu.SemaphoreType.DMA,
            pltpu.SemaphoreType.DMA,
            pltpu.SemaphoreType.DMA,
        ],
        compiler_params=_SC_PARAMS,
    )
    return f(src_p, dst_p, xwst)


def _summ_body(h_ref, p_ref, z_ref, *, n_rows):
    p = p_ref[...]
    nrm = jnp.sqrt(jnp.sum(p * p))
    pn = p / (nrm + 1e-8)
    h = h_ref[...]
    y = lax.dot_general(pn, h, (((1,), (1,)), ((), ())),
                        preferred_element_type=_f32)
    iota = lax.broadcasted_iota(jnp.int32, (1, n_rows), 1)

    def body(k, y):
        m = jnp.max(y)
        idx = jnp.min(jnp.where(y == m, iota, n_rows))
        row = h_ref[pl.ds(idx, 1), :]
        z_ref[pl.ds(k, 1), :] = row * jnp.tanh(m)
        return jnp.where(iota == idx, -jnp.inf, y)

    lax.fori_loop(0, TOPK, body, y)


def _summarize(h, p):
    d = h.shape[1]
    return pl.pallas_call(
        functools.partial(_summ_body, n_rows=h.shape[0]),
        out_shape=jax.ShapeDtypeStruct((TOPK, d), _f32),
    )(h, p[None, :])


def _mv_body(wih_ref, whh_ref, z_ref, p_ref, gi_ref, gh_ref):
    gi_ref[...] = lax.dot_general(wih_ref[...], z_ref[...], (((1,), (0,)), ((), ())),
                                  preferred_element_type=_f32)
    gh_ref[...] = lax.dot_general(whh_ref[...], p_ref[...], (((1,), (0,)), ((), ())),
                                  preferred_element_type=_f32)


def _gru_matvec(Wih, Whh, z2, p2, blk):
    r3 = Wih.shape[0]
    gin = Wih.shape[1]
    np_ = Whh.shape[1]
    gi, gh = pl.pallas_call(
        _mv_body,
        grid=(r3 // blk,),
        in_specs=[
            pl.BlockSpec((blk, gin), lambda i: (i, 0)),
            pl.BlockSpec((blk, np_), lambda i: (i, 0)),
            pl.BlockSpec((gin, 1), lambda i: (0, 0)),
            pl.BlockSpec((np_, 1), lambda i: (0, 0)),
        ],
        out_specs=[
            pl.BlockSpec((blk, 1), lambda i: (i, 0)),
            pl.BlockSpec((blk, 1), lambda i: (i, 0)),
        ],
        out_shape=[
            jax.ShapeDtypeStruct((r3, 1), _f32),
            jax.ShapeDtypeStruct((r3, 1), _f32),
        ],
    )(Wih, Whh, z2, p2)
    return gi, gh


def _gate_body(gi_ref, gh_ref, prm_ref, bih_ref, bhh_ref, out_ref):
    gi = gi_ref[...] + bih_ref[...]
    gh = gh_ref[...] + bhh_ref[...]
    r = jax.nn.sigmoid(gi[0:1] + gh[0:1])
    zz = jax.nn.sigmoid(gi[1:2] + gh[1:2])
    n = jnp.tanh(gi[2:3] + r * gh[2:3])
    out_ref[...] = (1.0 - zz) * n + zz * prm_ref[...]


def _gru_gate(gi, gh, prm, bih, bhh):
    np_ = prm.shape[0]
    return pl.pallas_call(
        _gate_body,
        out_shape=jax.ShapeDtypeStruct((1, np_), _f32),
    )(gi.reshape(3, np_), gh.reshape(3, np_), prm[None, :],
      bih.reshape(3, np_), bhh.reshape(3, np_))


def _xw_body(h_ref, w_ref, dinvt_ref, o_ref):
    o_ref[...] = lax.dot_general(w_ref[...], h_ref[...], (((0,), (1,)), ((), ())),
                                 preferred_element_type=_f32) * dinvt_ref[...]


def _xw_scaled_t(h, W, dinvt):
    din = h.shape[1]
    rblk = 2048
    return pl.pallas_call(
        _xw_body,
        grid=(NPAD // rblk,),
        in_specs=[
            pl.BlockSpec((rblk, din), lambda i: (i, 0)),
            pl.BlockSpec((din, DH), lambda i: (0, 0)),
            pl.BlockSpec((1, rblk), lambda i: (0, i)),
        ],
        out_specs=pl.BlockSpec((DH, rblk), lambda i: (0, i)),
        out_shape=jax.ShapeDtypeStruct((DH, NPAD), _f32),
    )(h, W, dinvt)


def _combine_body(acc_ref, xwst_ref, dinvt_ref, b_ref, o_ref, *, relu):
    t = ((acc_ref[0] + acc_ref[1]) + (acc_ref[2] + acc_ref[3])
         + xwst_ref[...]) * dinvt_ref[...] + b_ref[...]
    if relu:
        t = jnp.maximum(t, 0.0)
    o_ref[...] = t.T


def _combine(acct, xwst, dinvt, bcol, relu):
    rblk = 2048
    return pl.pallas_call(
        functools.partial(_combine_body, relu=relu),
        grid=(NPAD // rblk,),
        in_specs=[
            pl.BlockSpec((NM, DH, rblk), lambda i: (0, 0, i)),
            pl.BlockSpec((DH, rblk), lambda i: (0, i)),
            pl.BlockSpec((1, rblk), lambda i: (0, i)),
            pl.BlockSpec((DH, 1), lambda i: (0, 0)),
        ],
        out_specs=pl.BlockSpec((rblk, DH), lambda i: (i, 0)),
        out_shape=jax.ShapeDtypeStruct((N, DH), _f32),
    )(acct, xwst, dinvt, bcol)


def _layer(h, p, W, b, Wih, Whh, bih, bhh, src_p, dst_p, dinvt, blk, relu):
    Z = _summarize(h, p)
    z2 = Z.T.reshape(-1, 1)
    prm = jnp.concatenate([W.reshape(-1), b])
    gi, gh = _gru_matvec(Wih, Whh, z2, prm[:, None], blk)
    new = _gru_gate(gi, gh, prm, bih, bhh)[0]
    Wn = new[: W.shape[0] * W.shape[1]].reshape(W.shape)
    bn = new[W.shape[0] * W.shape[1]:]
    xwst = _xw_scaled_t(h, Wn, dinvt)
    acct = _edge_scatter(src_p, dst_p, xwst)
    return _combine(acct, xwst, dinvt, bn[:, None], relu)


def kernel(x, edge_index, W0, b0, W1, b1, p0, p1,
           Wih0, Whh0, bih0, bhh0, Wih1, Whh1, bih1, bhh1):
    pad = jnp.full((E_PAD - E,), N, jnp.int32)
    src_p = jnp.concatenate([edge_index[0], pad])
    dst_p = jnp.concatenate([edge_index[1], pad])
    degp = _degree(dst_p)
    dinv = lax.rsqrt(jnp.sum(degp, axis=0) + 1.0)
    dinvt = dinv[None, :]
    h1 = _layer(x, p0, W0, b0, Wih0, Whh0, bih0, bhh0,
                src_p, dst_p, dinvt, 688, relu=True)
    out = _layer(h1, p1, W1, b1, Wih1, Whh1, bih1, bhh1,
                 src_p, dst_p, dinvt, 1056, relu=False)
    return out


# speedup vs baseline: 1.0267x; 1.0267x over previous
import functools

import jax
import jax.numpy as jnp
from jax import lax
from jax.experimental import pallas as pl
from jax.experimental.pallas import tpu as pltpu
from jax.experimental.pallas import tpu_sc as plsc

N = 10000
E = 320000
DIN = 128
DH = 32
DOUT = 32
TOPK = 16
NP0 = DIN * DH + DH
NP1 = DH * DOUT + DOUT

NC = 2
NS = 16
NW = NC * NS
NPAD = 10240
SLICE = NPAD // NS
E_PAD = 327680
E_PER_W = E_PAD // NW
CHUNK = 2048
N_CH = E_PER_W // CHUNK
SCHUNK = 1280
S_CH = E_PER_W // SCHUNK

_f32 = jnp.float32


def _mesh():
    return plsc.VectorSubcoreMesh(
        core_axis_name="c", subcore_axis_name="s", num_cores=NC, num_subcores=NS
    )


_SC_PARAMS = pltpu.CompilerParams(use_tc_tiling_on_sc=False, needs_layout_passes=False)


_Z16 = None


def _deg_body(dst_hbm, out_hbm, idx_v, hist_v):
    c = lax.axis_index("c")
    s = lax.axis_index("s")
    wid = s * NC + c
    zeros16 = jnp.zeros((16,), _f32)
    ones16 = jnp.ones((16,), _f32)

    @plsc.parallel_loop(0, NPAD, 16, unroll=8)
    def zbody(i):
        hist_v[pl.ds(i, 16)] = zeros16

    for k in range(N_CH):
        base = pl.multiple_of(wid * E_PER_W + k * CHUNK, 8)
        pltpu.sync_copy(dst_hbm.at[pl.ds(base, CHUNK)], idx_v)

        @plsc.parallel_loop(0, CHUNK, 16, unroll=8)
        def ibody(i):
            d = idx_v[pl.ds(i, 16)]
            plsc.addupdate_scatter(hist_v, [d], ones16)

    pltpu.sync_copy(hist_v, out_hbm.at[wid])


def _degree(dst_p):
    f = pl.kernel(
        _deg_body,
        out_type=jax.ShapeDtypeStruct((NW, NPAD), _f32),
        mesh=_mesh(),
        scratch_types=[
            pltpu.VMEM((CHUNK,), jnp.int32),
            pltpu.VMEM((NPAD,), _f32),
        ],
        compiler_params=_SC_PARAMS,
    )
    return f(dst_p)


E_HALF = E_PAD // NC
ECH = 16384
E_NCH = E_HALF // ECH


def _edge_body(src_hbm, dst_hbm, xwst_hbm, out_hbm,
               sidx_a, sidx_b, didx_a, didx_b, feat0, feat1, acc0, acc1,
               sem0, sem1, sem2, sem3):
    c = lax.axis_index("c")
    s = lax.axis_index("s")
    zeros16 = jnp.zeros((16,), _f32)
    sidxs = (sidx_a, sidx_b)
    didxs = (didx_a, didx_b)
    sems = ((sem0, sem1), (sem2, sem3))

    def load(k, b):
        base = pl.multiple_of(c * E_HALF + k * ECH, 8)
        d1 = pltpu.async_copy(src_hbm.at[pl.ds(base, ECH)], sidxs[b], sems[b][0])
        d2 = pltpu.async_copy(dst_hbm.at[pl.ds(base, ECH)], didxs[b], sems[b][1])
        return (d1, d2)

    descs = [load(0, 0), None]

    @plsc.parallel_loop(0, NPAD, 16, unroll=8)
    def zbody(i):
        acc0[pl.ds(i, 16)] = zeros16
        acc1[pl.ds(i, 16)] = zeros16

    pltpu.sync_copy(xwst_hbm.at[2 * s], feat0)
    pltpu.sync_copy(xwst_hbm.at[2 * s + 1], feat1)
    for k in range(E_NCH):
        b = k & 1
        if k + 1 < E_NCH:
            descs[1 - b] = load(k + 1, 1 - b)
        for d in descs[b]:
            d.wait()
        sidx_v = sidxs[b]
        didx_v = didxs[b]

        @plsc.parallel_loop(0, ECH, 16, unroll=32)
        def ibody(i):
            sv = sidx_v[pl.ds(i, 16)]
            dv = didx_v[pl.ds(i, 16)]
            v0 = plsc.load_gather(feat0, [sv])
            plsc.addupdate_scatter(acc0, [dv], v0)
            v1 = plsc.load_gather(feat1, [sv])
            plsc.addupdate_scatter(acc1, [dv], v1)

    pltpu.sync_copy(acc0, out_hbm.at[c, 2 * s])
    pltpu.sync_copy(acc1, out_hbm.at[c, 2 * s + 1])


def _edge_scatter(src_p, dst_p, xwst):
    f = pl.kernel(
        _edge_body,
        out_type=jax.ShapeDtypeStruct((NC, DH, NPAD), _f32),
        mesh=_mesh(),
        scratch_types=[
            pltpu.VMEM((ECH,), jnp.int32),
            pltpu.VMEM((ECH,), jnp.int32),
            pltpu.VMEM((ECH,), jnp.int32),
            pltpu.VMEM((ECH,), jnp.int32),
            pltpu.VMEM((NPAD,), _f32),
            pltpu.VMEM((NPAD,), _f32),
            pltpu.VMEM((NPAD,), _f32),
            pltpu.VMEM((NPAD,), _f32),
            pltpu.SemaphoreType.DMA,
            pltpu.SemaphoreType.DMA,
            pltpu.SemaphoreType.DMA,
            pltpu.SemaphoreType.DMA,
        ],
        compiler_params=_SC_PARAMS,
    )
    return f(src_p, dst_p, xwst)


def _summ_body(h_ref, p_ref, z_ref, *, n_rows):
    p = p_ref[...]
    nrm = jnp.sqrt(jnp.sum(p * p))
    pn = p / (nrm + 1e-8)
    h = h_ref[...]
    y = lax.dot_general(pn, h, (((1,), (1,)), ((), ())),
                        preferred_element_type=_f32)
    iota = lax.broadcasted_iota(jnp.int32, (1, n_rows), 1)

    def body(k, y):
        m = jnp.max(y)
        idx = jnp.min(jnp.where(y == m, iota, n_rows))
        row = h_ref[pl.ds(idx, 1), :]
        z_ref[pl.ds(k, 1), :] = row * jnp.tanh(m)
        return jnp.where(iota == idx, -jnp.inf, y)

    lax.fori_loop(0, TOPK, body, y)


def _summarize(h, p):
    d = h.shape[1]
    return pl.pallas_call(
        functools.partial(_summ_body, n_rows=h.shape[0]),
        out_shape=jax.ShapeDtypeStruct((TOPK, d), _f32),
    )(h, p[None, :])


def _mv_body(wih_ref, whh_ref, z_ref, p_ref, gi_ref, gh_ref):
    gi_ref[...] = lax.dot_general(wih_ref[...], z_ref[...], (((1,), (0,)), ((), ())),
                                  preferred_element_type=_f32)
    gh_ref[...] = lax.dot_general(whh_ref[...], p_ref[...], (((1,), (0,)), ((), ())),
                                  preferred_element_type=_f32)


def _gru_matvec(Wih, Whh, z2, p2, blk):
    r3 = Wih.shape[0]
    gin = Wih.shape[1]
    np_ = Whh.shape[1]
    gi, gh = pl.pallas_call(
        _mv_body,
        grid=(r3 // blk,),
        in_specs=[
            pl.BlockSpec((blk, gin), lambda i: (i, 0)),
            pl.BlockSpec((blk, np_), lambda i: (i, 0)),
            pl.BlockSpec((gin, 1), lambda i: (0, 0)),
            pl.BlockSpec((np_, 1), lambda i: (0, 0)),
        ],
        out_specs=[
            pl.BlockSpec((blk, 1), lambda i: (i, 0)),
            pl.BlockSpec((blk, 1), lambda i: (i, 0)),
        ],
        out_shape=[
            jax.ShapeDtypeStruct((r3, 1), _f32),
            jax.ShapeDtypeStruct((r3, 1), _f32),
        ],
    )(Wih, Whh, z2, p2)
    return gi, gh


def _gate_body(gi_ref, gh_ref, prm_ref, bih_ref, bhh_ref, out_ref):
    gi = gi_ref[...] + bih_ref[...]
    gh = gh_ref[...] + bhh_ref[...]
    r = jax.nn.sigmoid(gi[0:1] + gh[0:1])
    zz = jax.nn.sigmoid(gi[1:2] + gh[1:2])
    n = jnp.tanh(gi[2:3] + r * gh[2:3])
    out_ref[...] = (1.0 - zz) * n + zz * prm_ref[...]


def _gru_gate(gi, gh, prm, bih, bhh):
    np_ = prm.shape[0]
    return pl.pallas_call(
        _gate_body,
        out_shape=jax.ShapeDtypeStruct((1, np_), _f32),
    )(gi.reshape(3, np_), gh.reshape(3, np_), prm[None, :],
      bih.reshape(3, np_), bhh.reshape(3, np_))


def _xw_body(h_ref, w_ref, dinvt_ref, o_ref):
    o_ref[...] = lax.dot_general(w_ref[...], h_ref[...], (((0,), (1,)), ((), ())),
                                 preferred_element_type=_f32) * dinvt_ref[...]


def _xw_scaled_t(h, W, dinvt):
    din = h.shape[1]
    rblk = 2048
    return pl.pallas_call(
        _xw_body,
        grid=(NPAD // rblk,),
        in_specs=[
            pl.BlockSpec((rblk, din), lambda i: (i, 0)),
            pl.BlockSpec((din, DH), lambda i: (0, 0)),
            pl.BlockSpec((1, rblk), lambda i: (0, i)),
        ],
        out_specs=pl.BlockSpec((DH, rblk), lambda i: (0, i)),
        out_shape=jax.ShapeDtypeStruct((DH, NPAD), _f32),
    )(h, W, dinvt)


def _combine_body(acc_ref, xwst_ref, dinvt_ref, b_ref, o_ref, *, relu):
    t = (acc_ref[0] + acc_ref[1] + xwst_ref[...]) * dinvt_ref[...] + b_ref[...]
    if relu:
        t = jnp.maximum(t, 0.0)
    o_ref[...] = t.T


def _combine(acct, xwst, dinvt, bcol, relu):
    rblk = 2048
    return pl.pallas_call(
        functools.partial(_combine_body, relu=relu),
        grid=(NPAD // rblk,),
        in_specs=[
            pl.BlockSpec((NC, DH, rblk), lambda i: (0, 0, i)),
            pl.BlockSpec((DH, rblk), lambda i: (0, i)),
            pl.BlockSpec((1, rblk), lambda i: (0, i)),
            pl.BlockSpec((DH, 1), lambda i: (0, 0)),
        ],
        out_specs=pl.BlockSpec((rblk, DH), lambda i: (i, 0)),
        out_shape=jax.ShapeDtypeStruct((N, DH), _f32),
    )(acct, xwst, dinvt, bcol)


def _layer(h, p, W, b, Wih, Whh, bih, bhh, src_p, dst_p, dinvt, blk, relu):
    Z = _summarize(h, p)
    z2 = Z.T.reshape(-1, 1)
    prm = jnp.concatenate([W.reshape(-1), b])
    gi, gh = _gru_matvec(Wih, Whh, z2, prm[:, None], blk)
    new = _gru_gate(gi, gh, prm, bih, bhh)[0]
    Wn = new[: W.shape[0] * W.shape[1]].reshape(W.shape)
    bn = new[W.shape[0] * W.shape[1]:]
    xwst = _xw_scaled_t(h, Wn, dinvt)
    acct = _edge_scatter(src_p, dst_p, xwst)
    return _combine(acct, xwst, dinvt, bn[:, None], relu)


def kernel(x, edge_index, W0, b0, W1, b1, p0, p1,
           Wih0, Whh0, bih0, bhh0, Wih1, Whh1, bih1, bhh1):
    pad = jnp.full((E_PAD - E,), N, jnp.int32)
    src_p = jnp.concatenate([edge_index[0], pad])
    dst_p = jnp.concatenate([edge_index[1], pad])
    degp = _degree(dst_p)
    dinv = lax.rsqrt(jnp.sum(degp, axis=0) + 1.0)
    dinvt = dinv[None, :]
    h1 = _layer(x, p0, W0, b0, Wih0, Whh0, bih0, bhh0,
                src_p, dst_p, dinvt, 688, relu=True)
    out = _layer(h1, p1, W1, b1, Wih1, Whh1, bih1, bhh1,
                 src_p, dst_p, dinvt, 1056, relu=False)
    return out


# speedup vs baseline: 1.0543x; 1.0269x over previous
import functools

import jax
import jax.numpy as jnp
from jax import lax
from jax.experimental import pallas as pl
from jax.experimental.pallas import tpu as pltpu
from jax.experimental.pallas import tpu_sc as plsc

N = 10000
E = 320000
DIN = 128
DH = 32
DOUT = 32
TOPK = 16
NP0 = DIN * DH + DH
NP1 = DH * DOUT + DOUT

NC = 2
NS = 16
NW = NC * NS
NPAD = 10240
SLICE = NPAD // NS
E_PAD = 327680
E_PER_W = E_PAD // NW
CHUNK = 2048
N_CH = E_PER_W // CHUNK
SCHUNK = 1280
S_CH = E_PER_W // SCHUNK

_f32 = jnp.float32


def _mesh():
    return plsc.VectorSubcoreMesh(
        core_axis_name="c", subcore_axis_name="s", num_cores=NC, num_subcores=NS
    )


_SC_PARAMS = pltpu.CompilerParams(use_tc_tiling_on_sc=False, needs_layout_passes=False)


_Z16 = None


def _deg_body(dst_hbm, out_hbm, idx_v, hist_v):
    c = lax.axis_index("c")
    s = lax.axis_index("s")
    wid = s * NC + c
    zeros16 = jnp.zeros((16,), _f32)
    ones16 = jnp.ones((16,), _f32)

    @plsc.parallel_loop(0, NPAD, 16, unroll=8)
    def zbody(i):
        hist_v[pl.ds(i, 16)] = zeros16

    for k in range(N_CH):
        base = pl.multiple_of(wid * E_PER_W + k * CHUNK, 8)
        pltpu.sync_copy(dst_hbm.at[pl.ds(base, CHUNK)], idx_v)

        @plsc.parallel_loop(0, CHUNK, 16, unroll=8)
        def ibody(i):
            d = idx_v[pl.ds(i, 16)]
            plsc.addupdate_scatter(hist_v, [d], ones16)

    pltpu.sync_copy(hist_v, out_hbm.at[wid])


def _degree(dst_p):
    f = pl.kernel(
        _deg_body,
        out_type=jax.ShapeDtypeStruct((NW, NPAD), _f32),
        mesh=_mesh(),
        scratch_types=[
            pltpu.VMEM((CHUNK,), jnp.int32),
            pltpu.VMEM((NPAD,), _f32),
        ],
        compiler_params=_SC_PARAMS,
    )
    return f(dst_p)


E_HALF = E_PAD // NC
ECH = 16384
E_NCH = E_HALF // ECH


def _edge_body(src_hbm, dst_hbm, xwst_hbm, out_hbm,
               sidx_a, sidx_b, didx_a, didx_b, feat0, feat1, acc0, acc1,
               sem0, sem1, sem2, sem3):
    c = lax.axis_index("c")
    s = lax.axis_index("s")
    zeros16 = jnp.zeros((16,), _f32)
    sidxs = (sidx_a, sidx_b)
    didxs = (didx_a, didx_b)
    sems = ((sem0, sem1), (sem2, sem3))

    def load(k, b):
        base = pl.multiple_of(c * E_HALF + k * ECH, 8)
        d1 = pltpu.async_copy(src_hbm.at[pl.ds(base, ECH)], sidxs[b], sems[b][0])
        d2 = pltpu.async_copy(dst_hbm.at[pl.ds(base, ECH)], didxs[b], sems[b][1])
        return (d1, d2)

    descs = [load(0, 0), None]

    @plsc.parallel_loop(0, NPAD, 16, unroll=8)
    def zbody(i):
        acc0[pl.ds(i, 16)] = zeros16
        acc1[pl.ds(i, 16)] = zeros16

    pltpu.sync_copy(xwst_hbm.at[2 * s], feat0)
    pltpu.sync_copy(xwst_hbm.at[2 * s + 1], feat1)
    for k in range(E_NCH):
        b = k & 1
        if k + 1 < E_NCH:
            descs[1 - b] = load(k + 1, 1 - b)
        for d in descs[b]:
            d.wait()
        sidx_v = sidxs[b]
        didx_v = didxs[b]

        @plsc.parallel_loop(0, ECH, 16, unroll=16)
        def ibody(i):
            sv = sidx_v[pl.ds(i, 16)]
            dv = didx_v[pl.ds(i, 16)]
            v0 = plsc.load_gather(feat0, [sv])
            plsc.addupdate_scatter(acc0, [dv], v0)
            v1 = plsc.load_gather(feat1, [sv])
            plsc.addupdate_scatter(acc1, [dv], v1)

    pltpu.sync_copy(acc0, out_hbm.at[c, 2 * s])
    pltpu.sync_copy(acc1, out_hbm.at[c, 2 * s + 1])


def _edge_scatter(src_p, dst_p, xwst):
    f = pl.kernel(
        _edge_body,
        out_type=jax.ShapeDtypeStruct((NC, DH, NPAD), _f32),
        mesh=_mesh(),
        scratch_types=[
            pltpu.VMEM((ECH,), jnp.int32),
            pltpu.VMEM((ECH,), jnp.int32),
            pltpu.VMEM((ECH,), jnp.int32),
            pltpu.VMEM((ECH,), jnp.int32),
            pltpu.VMEM((NPAD,), _f32),
            pltpu.VMEM((NPAD,), _f32),
            pltpu.VMEM((NPAD,), _f32),
            pltpu.VMEM((NPAD,), _f32),
            pltpu.SemaphoreType.DMA,
            pltpu.SemaphoreType.DMA,
            pltpu.SemaphoreType.DMA,
            pltpu.SemaphoreType.DMA,
        ],
        compiler_params=_SC_PARAMS,
    )
    return f(src_p, dst_p, xwst)


def _summ_body(h_ref, p_ref, z_ref, *, n_rows):
    p = p_ref[...]
    nrm = jnp.sqrt(jnp.sum(p * p))
    pn = p / (nrm + 1e-8)
    h = h_ref[...]
    y = lax.dot_general(pn, h, (((1,), (1,)), ((), ())),
                        preferred_element_type=_f32)
    iota = lax.broadcasted_iota(jnp.int32, (1, n_rows), 1)

    def body(k, y):
        m = jnp.max(y)
        idx = jnp.min(jnp.where(y == m, iota, n_rows))
        row = h_ref[pl.ds(idx, 1), :]
        z_ref[pl.ds(k, 1), :] = row * jnp.tanh(m)
        return jnp.where(iota == idx, -jnp.inf, y)

    lax.fori_loop(0, TOPK, body, y)


def _summarize(h, p):
    d = h.shape[1]
    return pl.pallas_call(
        functools.partial(_summ_body, n_rows=h.shape[0]),
        out_shape=jax.ShapeDtypeStruct((TOPK, d), _f32),
    )(h, p[None, :])


def _mv_body(wih_ref, whh_ref, z_ref, p_ref, gi_ref, gh_ref):
    gi_ref[...] = lax.dot_general(wih_ref[...], z_ref[...], (((1,), (0,)), ((), ())),
                                  preferred_element_type=_f32)
    gh_ref[...] = lax.dot_general(whh_ref[...], p_ref[...], (((1,), (0,)), ((), ())),
                                  preferred_element_type=_f32)


def _gru_matvec(Wih, Whh, z2, p2, blk):
    r3 = Wih.shape[0]
    gin = Wih.shape[1]
    np_ = Whh.shape[1]
    gi, gh = pl.pallas_call(
        _mv_body,
        grid=(r3 // blk,),
        in_specs=[
            pl.BlockSpec((blk, gin), lambda i: (i, 0)),
            pl.BlockSpec((blk, np_), lambda i: (i, 0)),
            pl.BlockSpec((gin, 1), lambda i: (0, 0)),
            pl.BlockSpec((np_, 1), lambda i: (0, 0)),
        ],
        out_specs=[
            pl.BlockSpec((blk, 1), lambda i: (i, 0)),
            pl.BlockSpec((blk, 1), lambda i: (i, 0)),
        ],
        out_shape=[
            jax.ShapeDtypeStruct((r3, 1), _f32),
            jax.ShapeDtypeStruct((r3, 1), _f32),
        ],
    )(Wih, Whh, z2, p2)
    return gi, gh


def _gate_body(gi_ref, gh_ref, prm_ref, bih_ref, bhh_ref, out_ref):
    gi = gi_ref[...] + bih_ref[...]
    gh = gh_ref[...] + bhh_ref[...]
    r = jax.nn.sigmoid(gi[0:1] + gh[0:1])
    zz = jax.nn.sigmoid(gi[1:2] + gh[1:2])
    n = jnp.tanh(gi[2:3] + r * gh[2:3])
    out_ref[...] = (1.0 - zz) * n + zz * prm_ref[...]


def _gru_gate(gi, gh, prm, bih, bhh):
    np_ = prm.shape[0]
    return pl.pallas_call(
        _gate_body,
        out_shape=jax.ShapeDtypeStruct((1, np_), _f32),
    )(gi.reshape(3, np_), gh.reshape(3, np_), prm[None, :],
      bih.reshape(3, np_), bhh.reshape(3, np_))


def _xw_body(h_ref, w_ref, dinvt_ref, o_ref):
    o_ref[...] = lax.dot_general(w_ref[...], h_ref[...], (((0,), (1,)), ((), ())),
                                 preferred_element_type=_f32) * dinvt_ref[...]


def _xw_scaled_t(h, W, dinvt):
    din = h.shape[1]
    rblk = 2048
    return pl.pallas_call(
        _xw_body,
        grid=(NPAD // rblk,),
        in_specs=[
            pl.BlockSpec((rblk, din), lambda i: (i, 0)),
            pl.BlockSpec((din, DH), lambda i: (0, 0)),
            pl.BlockSpec((1, rblk), lambda i: (0, i)),
        ],
        out_specs=pl.BlockSpec((DH, rblk), lambda i: (0, i)),
        out_shape=jax.ShapeDtypeStruct((DH, NPAD), _f32),
    )(h, W, dinvt)


def _combine_body(acc_ref, xwst_ref, dinvt_ref, b_ref, o_ref, *, relu):
    t = (acc_ref[0] + acc_ref[1] + xwst_ref[...]) * dinvt_ref[...] + b_ref[...]
    if relu:
        t = jnp.maximum(t, 0.0)
    o_ref[...] = t.T


def _combine(acct, xwst, dinvt, bcol, relu):
    rblk = 2048
    return pl.pallas_call(
        functools.partial(_combine_body, relu=relu),
        grid=(NPAD // rblk,),
        in_specs=[
            pl.BlockSpec((NC, DH, rblk), lambda i: (0, 0, i)),
            pl.BlockSpec((DH, rblk), lambda i: (0, i)),
            pl.BlockSpec((1, rblk), lambda i: (0, i)),
            pl.BlockSpec((DH, 1), lambda i: (0, 0)),
        ],
        out_specs=pl.BlockSpec((rblk, DH), lambda i: (i, 0)),
        out_shape=jax.ShapeDtypeStruct((N, DH), _f32),
    )(acct, xwst, dinvt, bcol)


def _layer(h, p, W, b, Wih, Whh, bih, bhh, src_p, dst_p, dinvt, blk, relu):
    Z = _summarize(h, p)
    z2 = Z.T.reshape(-1, 1)
    prm = jnp.concatenate([W.reshape(-1), b])
    gi, gh = _gru_matvec(Wih, Whh, z2, prm[:, None], blk)
    new = _gru_gate(gi, gh, prm, bih, bhh)[0]
    Wn = new[: W.shape[0] * W.shape[1]].reshape(W.shape)
    bn = new[W.shape[0] * W.shape[1]:]
    xwst = _xw_scaled_t(h, Wn, dinvt)
    acct = _edge_scatter(src_p, dst_p, xwst)
    return _combine(acct, xwst, dinvt, bn[:, None], relu)


def kernel(x, edge_index, W0, b0, W1, b1, p0, p1,
           Wih0, Whh0, bih0, bhh0, Wih1, Whh1, bih1, bhh1):
    pad = jnp.full((E_PAD - E,), N, jnp.int32)
    src_p = jnp.concatenate([edge_index[0], pad])
    dst_p = jnp.concatenate([edge_index[1], pad])
    degp = _degree(dst_p)
    dinv = lax.rsqrt(jnp.sum(degp, axis=0) + 1.0)
    dinvt = dinv[None, :]
    h1 = _layer(x, p0, W0, b0, Wih0, Whh0, bih0, bhh0,
                src_p, dst_p, dinvt, 688, relu=True)
    out = _layer(h1, p1, W1, b1, Wih1, Whh1, bih1, bhh1,
                 src_p, dst_p, dinvt, 1056, relu=False)
    return out
